# baseline (speedup 1.0000x reference)
import functools

import jax
import jax.numpy as jnp
from jax import lax
from jax.experimental import pallas as pl
from jax.experimental.pallas import tpu as pltpu
from jax.experimental.pallas import tpu_sc as plsc

_EPS = 1e-5


def _vq_block(x_ref, cb_ref, idx_ref, loss_ref, *, nblocks, inv_n):
    i = pl.program_id(0)
    xb = x_ref[...]
    cb = cb_ref[...]

    mean = jnp.mean(xb, axis=1, keepdims=True)
    var = jnp.mean((xb - mean) ** 2, axis=1, keepdims=True)
    xn = (xb - mean) / jnp.sqrt(var + _EPS)

    rn = jnp.sum(xn * xn, axis=1, keepdims=True)
    cn = jnp.sum(cb * cb, axis=1)
    dot = lax.dot_general(
        xn, cb, (((1,), (1,)), ((), ())),
        preferred_element_type=jnp.float32,
    )
    dist = rn - 2.0 * dot + cn[None, :]

    curT = dist[:, 0:128].T
    gbT = jnp.zeros(curT.shape, dtype=jnp.int32)
    for g in range(1, 8):
        ndT = dist[:, g * 128:(g + 1) * 128].T
        lt = ndT < curT
        gbT = jnp.where(lt, jnp.int32(g), gbT)
        curT = jnp.where(lt, ndT, curT)
    laneT = lax.broadcasted_iota(jnp.int32, curT.shape, 0)
    cidxT = gbT * 128 + laneT
    mindT = jnp.min(curT, axis=0, keepdims=True)
    selT = jnp.where(curT == mindT, cidxT, jnp.int32(2**30))
    idxT = jnp.min(selT, axis=0, keepdims=True)

    idx_ref[0, 0, :] = idxT[0]

    part = jnp.sum(mindT)

    @pl.when(i == 0)
    def _():
        loss_ref[0, 0] = 0.0

    loss_ref[0, 0] += part

    @pl.when(i == nblocks - 1)
    def _():
        loss_ref[0, 0] = loss_ref[0, 0] * inv_n


@functools.partial(jax.jit, static_argnames=("block_rows",))
def _vq_tc(flat, codebook, block_rows=2304):
    n, d = flat.shape
    k = codebook.shape[0]
    nblocks = n // block_rows
    idx, loss = pl.pallas_call(
        functools.partial(_vq_block, nblocks=nblocks, inv_n=1.0 / (n * d)),
        grid=(nblocks,),
        in_specs=[
            pl.BlockSpec((block_rows, d), lambda i: (i, 0)),
            pl.BlockSpec((k, d), lambda i: (0, 0)),
        ],
        out_specs=[
            pl.BlockSpec((1, 1, block_rows), lambda i: (i, 0, 0)),
            pl.BlockSpec(memory_space=pltpu.SMEM, block_shape=(1, 1),
                         index_map=lambda i: (0, 0)),
        ],
        out_shape=[
            jax.ShapeDtypeStruct((nblocks, 1, block_rows), jnp.int32),
            jax.ShapeDtypeStruct((1, 1), jnp.float32),
        ],
    )(flat, codebook)
    return idx, loss


def _make_sc_gather(v, dpad, b):
    info = plsc.get_sparse_core_info()
    nc, ns = info.num_cores, info.num_subcores
    nw = nc * ns
    bpw = b // nw
    mesh = plsc.VectorSubcoreMesh(core_axis_name="c", subcore_axis_name="s")

    @functools.partial(
        pl.kernel, mesh=mesh,
        out_type=jax.ShapeDtypeStruct((b, dpad), jnp.float32),
        scratch_types=[
            pltpu.VMEM((bpw,), jnp.int32),
            pltpu.VMEM((bpw, dpad), jnp.float32),
            pltpu.SemaphoreType.DMA,
        ],
    )
    def sc_gather(table_hbm, idx_hbm, out_hbm, idx_v, rows_v, sem):
        wid = lax.axis_index("s") * nc + lax.axis_index("c")
        base = wid * bpw
        pltpu.sync_copy(idx_hbm.at[pl.ds(base, bpw)], idx_v)
        pltpu.async_copy(table_hbm.at[idx_v], rows_v, sem).wait()
        pltpu.sync_copy(rows_v, out_hbm.at[pl.ds(base, bpw)])

    return sc_gather


def kernel(x, codebook):
    b, n, d = x.shape
    flat = x.reshape(-1, d)
    idx, loss = _vq_tc(flat, codebook)
    idx_flat = idx.reshape(-1)
    table = jnp.pad(codebook, ((0, 0), (0, 128 - d)))
    qp = _make_sc_gather(codebook.shape[0], 128, flat.shape[0])(table, idx_flat)
    return (qp[:, :d].reshape(b, n, d), idx.reshape(b, n), loss.reshape(()))


# speedup vs baseline: 1.2277x; 1.2277x over previous
import functools

import jax
import jax.numpy as jnp
from jax import lax
from jax.experimental import pallas as pl
from jax.experimental.pallas import tpu as pltpu

_EPS = 1e-5


def _vq_block(x_ref, cb_ref, cbhi_ref, cblo_ref, qT_ref, idx_ref, loss_ref,
              *, nblocks, inv_n):
    i = pl.program_id(0)
    xb = x_ref[...]
    cb = cb_ref[...]

    mean = jnp.mean(xb, axis=1, keepdims=True)
    var = jnp.mean((xb - mean) ** 2, axis=1, keepdims=True)
    xn = (xb - mean) / jnp.sqrt(var + _EPS)

    rn = jnp.sum(xn * xn, axis=1, keepdims=True)
    cn = jnp.sum(cb * cb, axis=1)
    dot = lax.dot_general(
        xn, cb, (((1,), (1,)), ((), ())),
        preferred_element_type=jnp.float32,
    )
    dist = rn - 2.0 * dot + cn[None, :]

    curT = dist[:, 0:128].T
    gbT = jnp.zeros(curT.shape, dtype=jnp.int32)
    for g in range(1, 8):
        ndT = dist[:, g * 128:(g + 1) * 128].T
        lt = ndT < curT
        gbT = jnp.where(lt, jnp.int32(g), gbT)
        curT = jnp.where(lt, ndT, curT)
    laneT = lax.broadcasted_iota(jnp.int32, curT.shape, 0)
    cidxT = gbT * 128 + laneT
    mindT = jnp.min(curT, axis=0, keepdims=True)
    selT = jnp.where(curT == mindT, cidxT, jnp.int32(2**30))
    idxT = jnp.min(selT, axis=0, keepdims=True)
    idsT = lax.broadcasted_iota(
        jnp.int32, (dist.shape[1], dist.shape[0]), 0)
    ohT16 = (idsT == idxT).astype(jnp.bfloat16)
    qT = lax.dot_general(
        cbhi_ref[...], ohT16, (((0,), (0,)), ((), ())),
        preferred_element_type=jnp.float32,
    ) + lax.dot_general(
        cblo_ref[...], ohT16, (((0,), (0,)), ((), ())),
        preferred_element_type=jnp.float32,
    )

    qT_ref[...] = qT
    idx_ref[0, 0, :] = idxT[0]

    part = jnp.sum(mindT)

    @pl.when(i == 0)
    def _():
        loss_ref[0, 0] = 0.0

    loss_ref[0, 0] += part

    @pl.when(i == nblocks - 1)
    def _():
        loss_ref[0, 0] = loss_ref[0, 0] * inv_n


@functools.partial(jax.jit, static_argnames=("block_rows",))
def _vq_tc(flat, codebook, block_rows=2304):
    n, d = flat.shape
    k = codebook.shape[0]
    nblocks = n // block_rows
    cb_hi = codebook.astype(jnp.bfloat16)
    cb_lo = (codebook - cb_hi.astype(jnp.float32)).astype(jnp.bfloat16)
    qT, idx, loss = pl.pallas_call(
        functools.partial(_vq_block, nblocks=nblocks, inv_n=1.0 / (n * d)),
        grid=(nblocks,),
        in_specs=[
            pl.BlockSpec((block_rows, d), lambda i: (i, 0)),
            pl.BlockSpec((k, d), lambda i: (0, 0)),
            pl.BlockSpec((k, d), lambda i: (0, 0)),
            pl.BlockSpec((k, d), lambda i: (0, 0)),
        ],
        out_specs=[
            pl.BlockSpec((d, block_rows), lambda i: (0, i)),
            pl.BlockSpec((1, 1, block_rows), lambda i: (i, 0, 0)),
            pl.BlockSpec(memory_space=pltpu.SMEM, block_shape=(1, 1),
                         index_map=lambda i: (0, 0)),
        ],
        out_shape=[
            jax.ShapeDtypeStruct((d, n), jnp.float32),
            jax.ShapeDtypeStruct((nblocks, 1, block_rows), jnp.int32),
            jax.ShapeDtypeStruct((1, 1), jnp.float32),
        ],
    )(flat, codebook, cb_hi, cb_lo)
    return qT, idx, loss


def kernel(x, codebook):
    b, n, d = x.shape
    flat = x.reshape(-1, d)
    qT, idx, loss = _vq_tc(flat, codebook)
    return (qT.T.reshape(b, n, d), idx.reshape(b, n), loss.reshape(()))
